# baseline (speedup 1.0000x reference)
import functools

import jax
import jax.numpy as jnp
from jax import lax
from jax.experimental import pallas as pl
from jax.experimental.pallas import tpu as pltpu
from jax.experimental.pallas import tpu_sc as plsc

D = 128
K = 10
NW = 32
G = 8
VPR = D // 16
NBUF = 4
GB = G * (K + 1) * D * 4
SB = G * D * 4


def _sc_aggregate(features, neigh_flat, nodes_pad, b_pad):
    q_per_w = b_pad // NW
    ng = q_per_w // G
    nq = ng // NBUF
    mesh = plsc.VectorSubcoreMesh(core_axis_name="c", subcore_axis_name="s")

    @functools.partial(
        pl.kernel,
        mesh=mesh,
        out_type=(
            jax.ShapeDtypeStruct((b_pad, D), jnp.float32),
            jax.ShapeDtypeStruct((b_pad, D), jnp.float32),
        ),
        scratch_types=[
            pltpu.VMEM((q_per_w * K,), jnp.int32),
            pltpu.VMEM((q_per_w,), jnp.int32),
            *[pltpu.VMEM((G * (K + 1), D), jnp.float32) for _ in range(NBUF)],
            *[pltpu.VMEM((G, D), jnp.float32) for _ in range(NBUF)],
            pltpu.SemaphoreType.DMA,
            *[pltpu.SemaphoreType.DMA for _ in range(NBUF)],
            *[pltpu.SemaphoreType.DMA for _ in range(NBUF)],
            *[pltpu.SemaphoreType.DMA for _ in range(NBUF)],
        ],
    )
    def agg(features_hbm, neigh_hbm, nodes_hbm, out_sum, out_self,
            idx_n, idx_s, *bufs):
        rows = list(bufs[0:NBUF])
        sum_v = list(bufs[NBUF:2 * NBUF])
        sem_i = bufs[2 * NBUF]
        sem_g = list(bufs[2 * NBUF + 1:2 * NBUF + 1 + NBUF])
        sem_w = list(bufs[2 * NBUF + 1 + NBUF:2 * NBUF + 1 + 2 * NBUF])
        sem_v = list(bufs[2 * NBUF + 1 + 2 * NBUF:2 * NBUF + 1 + 3 * NBUF])

        wid = lax.axis_index("s") * 2 + lax.axis_index("c")
        qbase = wid * q_per_w
        pltpu.async_copy(neigh_hbm.at[pl.ds(qbase * K, q_per_w * K)],
                         idx_n, sem_i).wait()
        pltpu.async_copy(nodes_hbm.at[pl.ds(qbase, q_per_w)],
                         idx_s, sem_i).wait()

        def gather(n, p):
            pltpu.async_copy(
                features_hbm.at[idx_n.at[pl.ds(n * (G * K), G * K)]],
                rows[p].at[pl.ds(0, G * K)], sem_g[p])
            pltpu.async_copy(
                features_hbm.at[idx_s.at[pl.ds(n * G, G)]],
                rows[p].at[pl.ds(G * K, G)], sem_g[p])

        for p in range(NBUF):
            gather(p, p)

        def ring_iter(i, carry):
            for p in range(NBUF):
                n = i * NBUF + p
                row0 = qbase + n * G
                pltpu.make_async_copy(
                    features_hbm.at[pl.ds(0, G * (K + 1))],
                    rows[p], sem_g[p]).wait()
                pltpu.async_copy(rows[p].at[pl.ds(G * K, G)],
                                 out_self.at[pl.ds(row0, G)], sem_v[p])
                @pl.when(i > 0)
                def _():
                    pltpu.make_async_copy(
                        sum_v[p], out_sum.at[pl.ds(qbase, G)],
                        sem_w[p]).wait()
                for q in range(G):
                    for v in range(VPR):
                        sl = pl.ds(v * 16, 16)
                        acc = rows[p][q * K, sl]
                        for k in range(1, K):
                            acc = acc + rows[p][q * K + k, sl]
                        sum_v[p][q, sl] = acc
                pltpu.make_async_copy(rows[p].at[pl.ds(G * K, G)],
                                      out_self.at[pl.ds(row0, G)],
                                      sem_v[p]).wait()
                @pl.when(i < nq - 1)
                def _():
                    gather(n + NBUF, p)
                pltpu.async_copy(sum_v[p], out_sum.at[pl.ds(row0, G)],
                                 sem_w[p])
            return carry

        lax.fori_loop(0, nq, ring_iter, 0)
        for p in range(NBUF):
            pltpu.make_async_copy(sum_v[p], out_sum.at[pl.ds(qbase, G)],
                                  sem_w[p]).wait()

    return agg(features, neigh_flat, nodes_pad)


def _tc_compress(sums, selfs, w, b_out):
    bt = 2048
    grid = (pl.cdiv(b_out, bt),)

    def body(w_ref, sum_ref, self_ref, o_ref):
        w_all = w_ref[...]
        wn = w_all[:, :D] * jnp.float32(1.0 / K)
        ws = w_all[:, D:]
        dn = (((1,), (1,)), ((), ()))
        o_ref[...] = jnp.maximum(
            lax.dot_general(wn, sum_ref[...], dn,
                            preferred_element_type=jnp.float32) +
            lax.dot_general(ws, self_ref[...], dn,
                            preferred_element_type=jnp.float32), 0.0)

    return pl.pallas_call(
        body,
        grid=grid,
        in_specs=[
            pl.BlockSpec((D, 2 * D), lambda j: (0, 0)),
            pl.BlockSpec((bt, D), lambda j: (j, 0)),
            pl.BlockSpec((bt, D), lambda j: (j, 0)),
        ],
        out_specs=pl.BlockSpec((D, bt), lambda j: (0, j)),
        out_shape=jax.ShapeDtypeStruct((D, b_out), jnp.float32),
    )(w, sums, selfs)


def kernel(nodes, neigh_idx, features, W_compress):
    b = nodes.shape[0]
    step = NW * G * NBUF
    b_pad = ((b + step - 1) // step) * step
    while ((b + 2047) // 2048) * 2048 > b_pad:
        b_pad += step
    neigh_flat = jnp.pad(neigh_idx.reshape(-1), (0, (b_pad - b) * K))
    nodes_pad = jnp.pad(nodes, (0, b_pad - b))
    sums, selfs = _sc_aggregate(features, neigh_flat, nodes_pad, b_pad)
    return _tc_compress(sums, selfs, W_compress, b)


# speedup vs baseline: 1.9344x; 1.9344x over previous
import functools

import jax
import jax.numpy as jnp
from jax import lax
from jax.experimental import pallas as pl
from jax.experimental.pallas import tpu as pltpu
from jax.experimental.pallas import tpu_sc as plsc

D = 128
K = 10
NW = 32
G = 8
VPR = D // 16
NBUF = 4
GB = G * (K + 1) * D * 4
SB = G * D * 4


def _sc_aggregate(features, neigh_flat, nodes_pad, b_pad):
    q_per_w = b_pad // NW
    ng = q_per_w // G
    nq = ng // NBUF
    mesh = plsc.VectorSubcoreMesh(core_axis_name="c", subcore_axis_name="s")

    @functools.partial(
        pl.kernel,
        mesh=mesh,
        out_type=(
            jax.ShapeDtypeStruct((b_pad, D), jnp.float32),
            jax.ShapeDtypeStruct((b_pad, D), jnp.float32),
        ),
        scratch_types=[
            pltpu.VMEM((q_per_w * K,), jnp.int32),
            pltpu.VMEM((q_per_w,), jnp.int32),
            *[pltpu.VMEM((G * (K + 1), D), jnp.float32) for _ in range(NBUF)],
            *[pltpu.VMEM((G, D), jnp.float32) for _ in range(NBUF)],
            pltpu.SemaphoreType.DMA,
            *[pltpu.SemaphoreType.DMA for _ in range(NBUF)],
            *[pltpu.SemaphoreType.DMA for _ in range(NBUF)],
            *[pltpu.SemaphoreType.DMA for _ in range(NBUF)],
        ],
    )
    def agg(features_hbm, neigh_hbm, nodes_hbm, out_sum, out_self,
            idx_n, idx_s, *bufs):
        rows = list(bufs[0:NBUF])
        sum_v = list(bufs[NBUF:2 * NBUF])
        sem_i = bufs[2 * NBUF]
        sem_g = list(bufs[2 * NBUF + 1:2 * NBUF + 1 + NBUF])
        sem_w = list(bufs[2 * NBUF + 1 + NBUF:2 * NBUF + 1 + 2 * NBUF])
        sem_v = list(bufs[2 * NBUF + 1 + 2 * NBUF:2 * NBUF + 1 + 3 * NBUF])

        wid = lax.axis_index("s") * 2 + lax.axis_index("c")
        qbase = wid * q_per_w
        pltpu.async_copy(neigh_hbm.at[pl.ds(qbase * K, q_per_w * K)],
                         idx_n, sem_i).wait()
        pltpu.async_copy(nodes_hbm.at[pl.ds(qbase, q_per_w)],
                         idx_s, sem_i).wait()

        def gather(n, p):
            pltpu.async_copy(
                features_hbm.at[idx_n.at[pl.ds(n * (G * K), G * K)]],
                rows[p].at[pl.ds(0, G * K)], sem_g[p])
            pltpu.async_copy(
                features_hbm.at[idx_s.at[pl.ds(n * G, G)]],
                rows[p].at[pl.ds(G * K, G)], sem_g[p])

        for p in range(NBUF):
            gather(p, p)

        def ring_iter(i, carry):
            for p in range(NBUF):
                n = i * NBUF + p
                row0 = qbase + n * G
                pltpu.make_async_copy(
                    features_hbm.at[pl.ds(0, G * (K + 1))],
                    rows[p], sem_g[p]).wait()
                pltpu.async_copy(rows[p].at[pl.ds(G * K, G)],
                                 out_self.at[pl.ds(row0, G)], sem_v[p])
                @pl.when(i > 0)
                def _():
                    pltpu.make_async_copy(
                        sum_v[p], out_sum.at[pl.ds(qbase, G)],
                        sem_w[p]).wait()
                for q in range(G):
                    for v in range(VPR):
                        sl = pl.ds(v * 16, 16)
                        acc = rows[p][q * K, sl]
                        for k in range(1, K):
                            acc = acc + rows[p][q * K + k, sl]
                        sum_v[p][q, sl] = acc
                pltpu.make_async_copy(rows[p].at[pl.ds(G * K, G)],
                                      out_self.at[pl.ds(row0, G)],
                                      sem_v[p]).wait()
                @pl.when(i < nq - 1)
                def _():
                    gather(n + NBUF, p)
                pltpu.async_copy(sum_v[p], out_sum.at[pl.ds(row0, G)],
                                 sem_w[p])
            return carry

        lax.fori_loop(0, nq, ring_iter, 0)
        for p in range(NBUF):
            pltpu.make_async_copy(sum_v[p], out_sum.at[pl.ds(qbase, G)],
                                  sem_w[p]).wait()

    return agg(features, neigh_flat, nodes_pad)


def _tc_compress(sums, selfs, w, b_out):
    bt = 1024
    grid = (pl.cdiv(b_out, bt),)

    def body(w_ref, sum_ref, self_ref, o_ref):
        w_all = w_ref[...]
        wn = w_all[:, :D] * jnp.float32(1.0 / K)
        ws = w_all[:, D:]
        dn = (((1,), (1,)), ((), ()))
        o_ref[...] = jnp.maximum(
            lax.dot_general(wn, sum_ref[...], dn,
                            preferred_element_type=jnp.float32) +
            lax.dot_general(ws, self_ref[...], dn,
                            preferred_element_type=jnp.float32), 0.0)

    return pl.pallas_call(
        body,
        grid=grid,
        in_specs=[
            pl.BlockSpec((D, 2 * D), lambda j: (0, 0)),
            pl.BlockSpec((bt, D), lambda j: (j, 0)),
            pl.BlockSpec((bt, D), lambda j: (j, 0)),
        ],
        out_specs=pl.BlockSpec((D, bt), lambda j: (0, j)),
        out_shape=jax.ShapeDtypeStruct((D, b_out), jnp.float32),
    )(w, sums, selfs)


def kernel(nodes, neigh_idx, features, W_compress):
    b = nodes.shape[0]
    step = NW * G * NBUF
    b_pad = ((b + step - 1) // step) * step
    while ((b + 1023) // 1024) * 1024 > b_pad:
        b_pad += step
    neigh_flat = jnp.pad(neigh_idx.reshape(-1), (0, (b_pad - b) * K))
    nodes_pad = jnp.pad(nodes, (0, b_pad - b))
    sums, selfs = _sc_aggregate(features, neigh_flat, nodes_pad, b_pad)
    return _tc_compress(sums, selfs, W_compress, b)
